# baseline (speedup 1.0000x reference)
import functools
import math

import jax
import jax.numpy as jnp
from jax import lax
from jax.experimental import pallas as pl
from jax.experimental.pallas import tpu as pltpu
from jax.experimental.pallas import tpu_sc as plsc

D_M = 64
SCALE = math.sqrt(D_M)
LANES = 16
XW = 128

try:
    _info = plsc.get_sparse_core_info()
    NC, NS = _info.num_cores, _info.num_subcores
except Exception:
    NC, NS = 2, 16
NW = NC * NS

NBUF = 4
PF = 2


def _emb_body(rows_per_w,
              x_hbm, table_hbm, out_hbm, idx_all, rows_v, rows2, sem_in,
              sem_out):
    wid = lax.axis_index("s") * NC + lax.axis_index("c")
    row0 = wid * rows_per_w
    prow0 = wid * rows_per_w * (XW // 2)

    def gather(g, b):
        return pltpu.make_async_copy(
            table_hbm.at[idx_all.at[g]], rows_v.at[b], sem_in.at[b])

    def write(g, b):
        return pltpu.make_async_copy(
            rows2.at[b], out_hbm.at[pl.ds(prow0 + g * (XW // 2), XW // 2)],
            sem_out.at[b])

    pltpu.sync_copy(x_hbm.at[pl.ds(row0, rows_per_w)], idx_all)
    for b in range(PF):
        gather(b, b).start()

    @pl.loop(0, rows_per_w, step=NBUF)
    def _outer(g0):
        for b in range(NBUF):
            g = g0 + b

            @pl.when(g + PF < rows_per_w)
            def _pf():
                gather(g + PF, (b + PF) % NBUF).start()

            gather(g, b).wait()

            @pl.when(g - NBUF >= 0)
            def _drain():
                write(g - NBUF, b).wait()

            @pl.loop(0, XW // 2, unroll=4)
            def _srow(r2):
                for j in range(D_M // LANES):
                    lo = (2 * r2, pl.ds(j * LANES, LANES))
                    hi = (2 * r2 + 1, pl.ds(j * LANES, LANES))
                    rows2[(b, r2, pl.ds(j * LANES, LANES))] = \
                        rows_v[(b, *lo)] * SCALE
                    rows2[(b, r2, pl.ds(D_M + j * LANES, LANES))] = \
                        rows_v[(b, *hi)] * SCALE

            write(g, b).start()

    for b in range(NBUF):
        write(rows_per_w - NBUF + b, (rows_per_w - NBUF + b) % NBUF).wait()


def _emb_lookup(xm, table):
    n_xrows = xm.shape[0]
    rows_per_w = n_xrows // NW
    n_prows = n_xrows * (XW // 2)

    mesh = plsc.VectorSubcoreMesh(core_axis_name="c", subcore_axis_name="s")
    body = functools.partial(_emb_body, rows_per_w)
    return pl.kernel(
        body,
        out_type=jax.ShapeDtypeStruct((n_prows, 2 * D_M), jnp.float32),
        mesh=mesh,
        compiler_params=pltpu.CompilerParams(use_tc_tiling_on_sc=False),
        scratch_types=[
            pltpu.VMEM((rows_per_w, XW), jnp.int32),
            pltpu.VMEM((NBUF, XW, D_M), jnp.float32),
            pltpu.VMEM((NBUF, XW // 2, 2 * D_M), jnp.float32),
            pltpu.SemaphoreType.DMA((NBUF,)),
            pltpu.SemaphoreType.DMA((NBUF,)),
        ],
    )(xm, table)


def kernel(x, table):
    B0, S = x.shape
    xm = x.astype(jnp.int32).reshape(B0 * S // XW, XW)
    out_k = _emb_lookup(xm, table)
    return out_k.reshape(B0, S, D_M)
